# baseline (speedup 1.0000x reference)
import jax
import jax.numpy as jnp
from jax import lax
from jax.experimental import pallas as pl
from jax.experimental.pallas import tpu as pltpu
from jax.experimental.pallas import tpu_sc as plsc

D = 64
B = 16384
NC, NS = 2, 16
NW = NC * NS
BPW = B // NW
L = 16
CH = 128
NCH = BPW // CH
W2 = 2 * D

N_ENT, N_REL = 1000000, 1000
TCH = 256
NPAD = 1000064
EBLK = (N_ENT + TCH - 1) // TCH
LAST_OFF = NPAD - TCH
ENT2_ROWS = (LAST_OFF + TCH) // 2
RPAD = 1024
RBLK = RPAD // TCH
REL2_ROWS = RPAD // 2
REL_W = 8
NSUB = (D // L) * (TCH // L)


def _transpose_body(ent_t, rel_t, ent2, rel2, in_v, st_v, m_v, isems,
                    osems):
    wid = lax.axis_index("s") * NC + lax.axis_index("c")

    nent = (EBLK - wid + NW - 1) // NW
    has_rel = jnp.logical_and(wid >= REL_W, wid < REL_W + RBLK)
    nunits = nent + has_rel.astype(jnp.int32)

    def issue(i):
        slot = lax.rem(i, 2)
        is_ent = i < nent

        @pl.when(is_ent)
        def _():
            off = pl.multiple_of(
                jnp.minimum((wid + i * NW) * TCH, LAST_OFF), CH)
            pltpu.async_copy(ent_t.at[:, pl.ds(off, TCH)],
                             in_v.at[pl.ds(slot * D, D), :], isems.at[slot])

        @pl.when(jnp.logical_not(is_ent))
        def _():
            off = pl.multiple_of((wid - REL_W) * TCH, CH)
            pltpu.async_copy(rel_t.at[:, pl.ds(off, TCH)],
                             in_v.at[pl.ds(slot * D, D), :], isems.at[slot])

    def drain_in(i):
        slot = lax.rem(i, 2)
        pltpu.make_async_copy(ent_t.at[:, pl.ds(0, CH)],
                              in_v.at[pl.ds(slot * D, D), :], isems.at[slot]).wait()

    issue(0)

    lanes = jnp.arange(L, dtype=jnp.int32)

    def body(i, carry):
        @pl.when(i + 1 < nunits)
        def _():
            issue(i + 1)

        drain_in(i)
        slot = lax.rem(i, 2)

        @pl.when(i >= 2)
        def _():
            pltpu.make_async_copy(st_v.at[pl.ds(slot * (TCH // 2), TCH // 2), :],
                                  ent2.at[pl.ds(0, TCH // 2), :],
                                  osems.at[slot]).wait()

        so = slot * D
        sto = slot * (TCH // 2)
        mo = slot * (NSUB * L * L)

        @plsc.parallel_loop(0, NSUB, 1, unroll=8)
        def _trA(sub):
            dc = lax.div(sub, TCH // L)
            ec = lax.rem(sub, TCH // L)
            mbase = mo + sub * (L * L)
            for i in range(L):
                vals = in_v[so + dc * L + i,
                            pl.ds(pl.multiple_of(ec * L, L), L)]
                skew = i * L + jnp.bitwise_and(lanes + i, L - 1)
                plsc.store_scatter(m_v, [mbase + skew], vals)

        @plsc.parallel_loop(0, NSUB, 1, unroll=8)
        def _trB(sub):
            dc = lax.div(sub, TCH // L)
            ec = lax.rem(sub, TCH // L)
            mbase = mo + sub * (L * L)
            for j in range(L):
                rd = lanes * L + jnp.bitwise_and(lanes + j, L - 1)
                g = plsc.load_gather(m_v, [mbase + rd])
                st_v[sto + ec * (L // 2) + (j >> 1),
                     pl.ds(pl.multiple_of((j & 1) * D + dc * L, L), L)] = g

        is_ent = i < nent

        @pl.when(is_ent)
        def _():
            dst = pl.multiple_of(
                jnp.minimum((wid + i * NW) * TCH, LAST_OFF) // 2, D)
            pltpu.async_copy(st_v.at[pl.ds(sto, TCH // 2), :],
                             ent2.at[pl.ds(dst, TCH // 2), :], osems.at[slot])

        @pl.when(jnp.logical_not(is_ent))
        def _():
            dst = pl.multiple_of((wid - REL_W) * (TCH // 2), D)
            pltpu.async_copy(st_v.at[pl.ds(sto, TCH // 2), :],
                             rel2.at[pl.ds(dst, TCH // 2), :], osems.at[slot])

        return carry

    lax.fori_loop(0, nunits, body, 0)

    pltpu.make_async_copy(st_v.at[pl.ds(0, TCH // 2), :],
                          ent2.at[pl.ds(0, TCH // 2), :],
                          osems.at[lax.rem(nunits - 1, 2)]).wait()

    @pl.when(nunits >= 2)
    def _():
        pltpu.make_async_copy(st_v.at[pl.ds(0, TCH // 2), :],
                              ent2.at[pl.ds(0, TCH // 2), :],
                              osems.at[lax.rem(nunits, 2)]).wait()


def _score_body(head_hbm, rel_hbm, tail_hbm, ent_hbm, relemb_hbm, out_hbm,
                hidx_v, ridx_v, tidx_v, hrow_v, rrow_v, trow_v,
                hbuf_v, rbuf_v, tbuf_v, q_v, out_v, sems):
    wid = lax.axis_index("s") * NC + lax.axis_index("c")
    base = wid * BPW

    pltpu.sync_copy(head_hbm.at[pl.ds(base, BPW)], hidx_v)
    pltpu.sync_copy(rel_hbm.at[pl.ds(base, BPW)], ridx_v)
    pltpu.sync_copy(tail_hbm.at[pl.ds(base, BPW)], tidx_v)

    for v in range(BPW // L):
        hrow_v[pl.ds(v * L, L)] = jnp.right_shift(hidx_v[pl.ds(v * L, L)], 1)
        rrow_v[pl.ds(v * L, L)] = jnp.right_shift(ridx_v[pl.ds(v * L, L)], 1)
        trow_v[pl.ds(v * L, L)] = jnp.right_shift(tidx_v[pl.ds(v * L, L)], 1)

    def issue(k):
        slot = lax.rem(k, 2)
        o = slot * CH
        pltpu.async_copy(ent_hbm.at[hrow_v.at[pl.ds(k * CH, CH)]],
                         hbuf_v.at[pl.ds(o, CH)], sems.at[slot])
        pltpu.async_copy(relemb_hbm.at[rrow_v.at[pl.ds(k * CH, CH)]],
                         rbuf_v.at[pl.ds(o, CH)], sems.at[slot])
        pltpu.async_copy(ent_hbm.at[trow_v.at[pl.ds(k * CH, CH)]],
                         tbuf_v.at[pl.ds(o, CH)], sems.at[slot])

    def drain(k):
        slot = lax.rem(k, 2)
        o = slot * CH
        pltpu.make_async_copy(ent_hbm.at[hrow_v.at[pl.ds(0, CH)]],
                              hbuf_v.at[pl.ds(o, CH)], sems.at[slot]).wait()
        pltpu.make_async_copy(ent_hbm.at[hrow_v.at[pl.ds(0, CH)]],
                              rbuf_v.at[pl.ds(o, CH)], sems.at[slot]).wait()
        pltpu.make_async_copy(ent_hbm.at[hrow_v.at[pl.ds(0, CH)]],
                              tbuf_v.at[pl.ds(o, CH)], sems.at[slot]).wait()

    issue(0)

    def chunk(k, carry):
        drain(k)

        @pl.when(k + 1 < NCH)
        def _():
            issue(k + 1)

        slot = lax.rem(k, 2)
        o = slot * CH

        def grp(gg, carry2):
            b0 = k * CH + gg * L
            vh = jnp.bitwise_and(hidx_v[pl.ds(b0, L)], 1) * D
            vr = jnp.bitwise_and(ridx_v[pl.ds(b0, L)], 1) * D
            vt = jnp.bitwise_and(tidx_v[pl.ds(b0, L)], 1) * D
            for j in range(L):
                i = gg * L + j
                hh, rh, th = vh[j], vr[j], vt[j]
                q = (hbuf_v[o + i, pl.ds(hh, L)] * rbuf_v[o + i, pl.ds(rh, L)]
                     * tbuf_v[o + i, pl.ds(th, L)])
                for c in range(L, D, L):
                    q += (hbuf_v[o + i, pl.ds(hh + c, L)]
                          * rbuf_v[o + i, pl.ds(rh + c, L)]
                          * tbuf_v[o + i, pl.ds(th + c, L)])
                q_v[pl.ds((b0 + j) * L, L)] = q
            return carry2

        lax.fori_loop(0, CH // L, grp, 0)
        return carry

    lax.fori_loop(0, NCH, chunk, 0)

    @plsc.parallel_loop(0, BPW // L, 1, unroll=2)
    def _rowB(g):
        rows = g * L + jnp.arange(L, dtype=jnp.int32)
        acc = plsc.load_gather(q_v, [rows * L])
        for l in range(1, L):
            acc += plsc.load_gather(q_v, [rows * L + l])
        out_v[pl.ds(g * L, L)] = acc

    pltpu.sync_copy(out_v, out_hbm.at[pl.ds(base, BPW)])


@jax.jit
def _distmult(head, relation, tail, ent_t, rel_t):
    mesh = plsc.VectorSubcoreMesh(core_axis_name="c", subcore_axis_name="s")
    ent2, rel2 = pl.kernel(
        _transpose_body,
        out_type=(jax.ShapeDtypeStruct((ENT2_ROWS, W2), jnp.float32),
                  jax.ShapeDtypeStruct((REL2_ROWS, W2), jnp.float32)),
        mesh=mesh,
        scratch_types=[
            pltpu.VMEM((2 * D, TCH), jnp.float32),
            pltpu.VMEM((2 * (TCH // 2), W2), jnp.float32),
            pltpu.VMEM((2 * NSUB * L * L,), jnp.float32),
            pltpu.SemaphoreType.DMA((2,)),
            pltpu.SemaphoreType.DMA((2,)),
        ],
        compiler_params=pltpu.CompilerParams(
            needs_layout_passes=False, use_tc_tiling_on_sc=True),
    )(ent_t, rel_t)

    return pl.kernel(
        _score_body,
        out_type=jax.ShapeDtypeStruct((B,), jnp.float32),
        mesh=mesh,
        scratch_types=[
            pltpu.VMEM((BPW,), jnp.int32),
            pltpu.VMEM((BPW,), jnp.int32),
            pltpu.VMEM((BPW,), jnp.int32),
            pltpu.VMEM((BPW,), jnp.int32),
            pltpu.VMEM((BPW,), jnp.int32),
            pltpu.VMEM((BPW,), jnp.int32),
            pltpu.VMEM((2 * CH, W2), jnp.float32),
            pltpu.VMEM((2 * CH, W2), jnp.float32),
            pltpu.VMEM((2 * CH, W2), jnp.float32),
            pltpu.VMEM((BPW * L,), jnp.float32),
            pltpu.VMEM((BPW,), jnp.float32),
            pltpu.SemaphoreType.DMA((2,)),
        ],
        compiler_params=pltpu.CompilerParams(
            needs_layout_passes=False, use_tc_tiling_on_sc=True),
    )(head, relation, tail, ent2, rel2)


def kernel(head, relation, tail, entity_emb, relation_emb):
    rel_t = jnp.pad(relation_emb.T, ((0, 0), (0, RPAD - N_REL)))
    return _distmult(head.astype(jnp.int32), relation.astype(jnp.int32),
                     tail.astype(jnp.int32), entity_emb.T, rel_t)


# speedup vs baseline: 1.6762x; 1.6762x over previous
import jax
import jax.numpy as jnp
from jax import lax
from jax.experimental import pallas as pl
from jax.experimental.pallas import tpu as pltpu
from jax.experimental.pallas import tpu_sc as plsc

D = 64
B = 16384
NC, NS = 2, 16
NW = NC * NS
BPW = B // NW
L = 16
CH = 128
NCH = BPW // CH
W2 = 2 * D
NSUB = (D // L) * (CH // L)

N_ENT, N_REL = 1000000, 1000
EBLK = (N_ENT + CH - 1) // CH
ENT2_ROWS = EBLK * D
RPAD = 1024
RBLK = RPAD // CH
REL2_ROWS = RPAD // 2
REL_W = 8


def _transpose_body(ent_t, rel_t, ent2, rel2, in_v, st_v, m_v, isems,
                    osems):
    wid = lax.axis_index("s") * NC + lax.axis_index("c")

    nent = (EBLK - wid + NW - 1) // NW
    has_rel = jnp.logical_and(wid >= REL_W, wid < REL_W + RBLK)
    nunits = nent + has_rel.astype(jnp.int32)

    def issue(i):
        slot = lax.rem(i, 3)
        is_ent = i < nent

        @pl.when(is_ent)
        def _():
            off = pl.multiple_of((wid + i * NW) * CH, CH)
            pltpu.async_copy(ent_t.at[:, pl.ds(off, CH)],
                             in_v.at[pl.ds(slot * D, D), :], isems.at[slot])

        @pl.when(jnp.logical_not(is_ent))
        def _():
            off = pl.multiple_of((wid - REL_W) * CH, CH)
            pltpu.async_copy(rel_t.at[:, pl.ds(off, CH)],
                             in_v.at[pl.ds(slot * D, D), :], isems.at[slot])

    def drain_in(i):
        slot = lax.rem(i, 3)
        pltpu.make_async_copy(ent_t.at[:, pl.ds(0, CH)],
                              in_v.at[pl.ds(slot * D, D), :], isems.at[slot]).wait()

    issue(0)

    @pl.when(nunits > 1)
    def _():
        issue(1)

    lanes = jnp.arange(L, dtype=jnp.int32)

    def body(i, carry):
        @pl.when(i + 2 < nunits)
        def _():
            issue(i + 2)

        drain_in(i)
        islot = lax.rem(i, 3)
        slot = lax.rem(i, 2)

        @pl.when(i >= 2)
        def _():
            pltpu.make_async_copy(st_v.at[pl.ds(slot * D, D), :],
                                  ent2.at[pl.ds(0, D), :],
                                  osems.at[slot]).wait()

        so = islot * D
        sto = slot * D
        mo = slot * (NSUB * L * L)

        @plsc.parallel_loop(0, NSUB, 1, unroll=8)
        def _trA(sub):
            dc = lax.div(sub, CH // L)
            ec = lax.rem(sub, CH // L)
            mbase = mo + sub * (L * L)
            for i in range(L):
                vals = in_v[so + dc * L + i,
                            pl.ds(pl.multiple_of(ec * L, L), L)]
                skew = i * L + jnp.bitwise_and(lanes + i, L - 1)
                plsc.store_scatter(m_v, [mbase + skew], vals)

        @plsc.parallel_loop(0, NSUB, 1, unroll=8)
        def _trB(sub):
            dc = lax.div(sub, CH // L)
            ec = lax.rem(sub, CH // L)
            mbase = mo + sub * (L * L)
            for j in range(L):
                rd = lanes * L + jnp.bitwise_and(lanes + j, L - 1)
                g = plsc.load_gather(m_v, [mbase + rd])
                st_v[sto + ec * (L // 2) + (j >> 1),
                     pl.ds(pl.multiple_of((j & 1) * D + dc * L, L), L)] = g

        is_ent = i < nent

        @pl.when(is_ent)
        def _():
            dst = pl.multiple_of((wid + i * NW) * D, D)
            pltpu.async_copy(st_v.at[pl.ds(slot * D, D), :],
                             ent2.at[pl.ds(dst, D), :], osems.at[slot])

        @pl.when(jnp.logical_not(is_ent))
        def _():
            dst = pl.multiple_of((wid - REL_W) * D, D)
            pltpu.async_copy(st_v.at[pl.ds(slot * D, D), :],
                             rel2.at[pl.ds(dst, D), :], osems.at[slot])

        return carry

    lax.fori_loop(0, nunits, body, 0)

    pltpu.make_async_copy(st_v.at[pl.ds(0, D), :], ent2.at[pl.ds(0, D), :],
                          osems.at[lax.rem(nunits - 1, 2)]).wait()

    @pl.when(nunits >= 2)
    def _():
        pltpu.make_async_copy(st_v.at[pl.ds(0, D), :],
                              ent2.at[pl.ds(0, D), :],
                              osems.at[lax.rem(nunits, 2)]).wait()


def _score_body(head_hbm, rel_hbm, tail_hbm, ent_hbm, relemb_hbm, out_hbm,
                hidx_v, ridx_v, tidx_v, hrow_v, rrow_v, trow_v,
                hbuf_v, rbuf_v, tbuf_v, q_v, out_v, sems):
    wid = lax.axis_index("s") * NC + lax.axis_index("c")
    base = wid * BPW

    pltpu.sync_copy(head_hbm.at[pl.ds(base, BPW)], hidx_v)
    pltpu.sync_copy(rel_hbm.at[pl.ds(base, BPW)], ridx_v)
    pltpu.sync_copy(tail_hbm.at[pl.ds(base, BPW)], tidx_v)

    for v in range(BPW // L):
        hrow_v[pl.ds(v * L, L)] = jnp.right_shift(hidx_v[pl.ds(v * L, L)], 1)
        rrow_v[pl.ds(v * L, L)] = jnp.right_shift(ridx_v[pl.ds(v * L, L)], 1)
        trow_v[pl.ds(v * L, L)] = jnp.right_shift(tidx_v[pl.ds(v * L, L)], 1)

    def issue(k):
        slot = lax.rem(k, 2)
        o = slot * CH
        pltpu.async_copy(ent_hbm.at[hrow_v.at[pl.ds(k * CH, CH)]],
                         hbuf_v.at[pl.ds(o, CH)], sems.at[slot])
        pltpu.async_copy(relemb_hbm.at[rrow_v.at[pl.ds(k * CH, CH)]],
                         rbuf_v.at[pl.ds(o, CH)], sems.at[slot])
        pltpu.async_copy(ent_hbm.at[trow_v.at[pl.ds(k * CH, CH)]],
                         tbuf_v.at[pl.ds(o, CH)], sems.at[slot])

    def drain(k):
        slot = lax.rem(k, 2)
        o = slot * CH
        pltpu.make_async_copy(ent_hbm.at[hrow_v.at[pl.ds(0, CH)]],
                              hbuf_v.at[pl.ds(o, CH)], sems.at[slot]).wait()
        pltpu.make_async_copy(ent_hbm.at[hrow_v.at[pl.ds(0, CH)]],
                              rbuf_v.at[pl.ds(o, CH)], sems.at[slot]).wait()
        pltpu.make_async_copy(ent_hbm.at[hrow_v.at[pl.ds(0, CH)]],
                              tbuf_v.at[pl.ds(o, CH)], sems.at[slot]).wait()

    issue(0)

    def chunk(k, carry):
        drain(k)

        @pl.when(k + 1 < NCH)
        def _():
            issue(k + 1)

        slot = lax.rem(k, 2)
        o = slot * CH

        def grp(gg, carry2):
            b0 = k * CH + gg * L
            vh = jnp.bitwise_and(hidx_v[pl.ds(b0, L)], 1) * D
            vr = jnp.bitwise_and(ridx_v[pl.ds(b0, L)], 1) * D
            vt = jnp.bitwise_and(tidx_v[pl.ds(b0, L)], 1) * D
            for j in range(L):
                i = gg * L + j
                hh, rh, th = vh[j], vr[j], vt[j]
                q = (hbuf_v[o + i, pl.ds(hh, L)] * rbuf_v[o + i, pl.ds(rh, L)]
                     * tbuf_v[o + i, pl.ds(th, L)])
                for c in range(L, D, L):
                    q += (hbuf_v[o + i, pl.ds(hh + c, L)]
                          * rbuf_v[o + i, pl.ds(rh + c, L)]
                          * tbuf_v[o + i, pl.ds(th + c, L)])
                q_v[pl.ds((b0 + j) * L, L)] = q
            return carry2

        lax.fori_loop(0, CH // L, grp, 0)
        return carry

    lax.fori_loop(0, NCH, chunk, 0)

    @plsc.parallel_loop(0, BPW // L, 1, unroll=2)
    def _rowB(g):
        rows = g * L + jnp.arange(L, dtype=jnp.int32)
        acc = plsc.load_gather(q_v, [rows * L])
        for l in range(1, L):
            acc += plsc.load_gather(q_v, [rows * L + l])
        out_v[pl.ds(g * L, L)] = acc

    pltpu.sync_copy(out_v, out_hbm.at[pl.ds(base, BPW)])


@jax.jit
def _distmult(head, relation, tail, ent_t, rel_t):
    mesh = plsc.VectorSubcoreMesh(core_axis_name="c", subcore_axis_name="s")
    ent2, rel2 = pl.kernel(
        _transpose_body,
        out_type=(jax.ShapeDtypeStruct((ENT2_ROWS, W2), jnp.float32),
                  jax.ShapeDtypeStruct((REL2_ROWS, W2), jnp.float32)),
        mesh=mesh,
        scratch_types=[
            pltpu.VMEM((3 * D, CH), jnp.float32),
            pltpu.VMEM((2 * D, W2), jnp.float32),
            pltpu.VMEM((2 * NSUB * L * L,), jnp.float32),
            pltpu.SemaphoreType.DMA((3,)),
            pltpu.SemaphoreType.DMA((2,)),
        ],
        compiler_params=pltpu.CompilerParams(
            needs_layout_passes=False, use_tc_tiling_on_sc=True),
    )(ent_t, rel_t)

    return pl.kernel(
        _score_body,
        out_type=jax.ShapeDtypeStruct((B,), jnp.float32),
        mesh=mesh,
        scratch_types=[
            pltpu.VMEM((BPW,), jnp.int32),
            pltpu.VMEM((BPW,), jnp.int32),
            pltpu.VMEM((BPW,), jnp.int32),
            pltpu.VMEM((BPW,), jnp.int32),
            pltpu.VMEM((BPW,), jnp.int32),
            pltpu.VMEM((BPW,), jnp.int32),
            pltpu.VMEM((2 * CH, W2), jnp.float32),
            pltpu.VMEM((2 * CH, W2), jnp.float32),
            pltpu.VMEM((2 * CH, W2), jnp.float32),
            pltpu.VMEM((BPW * L,), jnp.float32),
            pltpu.VMEM((BPW,), jnp.float32),
            pltpu.SemaphoreType.DMA((2,)),
        ],
        compiler_params=pltpu.CompilerParams(
            needs_layout_passes=False, use_tc_tiling_on_sc=True),
    )(head, relation, tail, ent2, rel2)


def kernel(head, relation, tail, entity_emb, relation_emb):
    rel_t = jnp.pad(relation_emb.T, ((0, 0), (0, RPAD - N_REL)))
    return _distmult(head.astype(jnp.int32), relation.astype(jnp.int32),
                     tail.astype(jnp.int32), entity_emb.T, rel_t)


# speedup vs baseline: 1.7830x; 1.0637x over previous
import jax
import jax.numpy as jnp
from jax import lax
from jax.experimental import pallas as pl
from jax.experimental.pallas import tpu as pltpu
from jax.experimental.pallas import tpu_sc as plsc

D = 64
B = 16384
NC, NS = 2, 16
NW = NC * NS
BPW = B // NW
L = 16
CH = 128
NCH = BPW // CH
W2 = 2 * D
NSUB = (D // L) * (CH // L)

N_ENT, N_REL = 1000000, 1000
EBLK = (N_ENT + CH - 1) // CH
ENT2_ROWS = EBLK * D
RPAD = 1024
RBLK = RPAD // CH
REL2_ROWS = RPAD // 2
REL_W = 8


def _transpose_body(ent_t, rel_t, ent2, rel2, in_v, st_v, m_v, isems,
                    osems):
    wid = lax.axis_index("s") * NC + lax.axis_index("c")

    nent = (EBLK - wid + NW - 1) // NW
    has_rel = jnp.logical_and(wid >= REL_W, wid < REL_W + RBLK)
    nunits = nent + has_rel.astype(jnp.int32)

    def issue(i):
        slot = lax.rem(i, 4)
        is_ent = i < nent

        @pl.when(is_ent)
        def _():
            off = pl.multiple_of((wid + i * NW) * CH, CH)
            pltpu.async_copy(ent_t.at[:, pl.ds(off, CH)],
                             in_v.at[pl.ds(slot * D, D), :], isems.at[slot])

        @pl.when(jnp.logical_not(is_ent))
        def _():
            off = pl.multiple_of((wid - REL_W) * CH, CH)
            pltpu.async_copy(rel_t.at[:, pl.ds(off, CH)],
                             in_v.at[pl.ds(slot * D, D), :], isems.at[slot])

    def drain_in(i):
        slot = lax.rem(i, 4)
        pltpu.make_async_copy(ent_t.at[:, pl.ds(0, CH)],
                              in_v.at[pl.ds(slot * D, D), :], isems.at[slot]).wait()

    issue(0)

    @pl.when(nunits > 1)
    def _():
        issue(1)

    @pl.when(nunits > 2)
    def _():
        issue(2)

    lanes = jnp.arange(L, dtype=jnp.int32)

    def body(i, carry):
        @pl.when(i + 3 < nunits)
        def _():
            issue(i + 3)

        drain_in(i)
        islot = lax.rem(i, 4)
        slot = lax.rem(i, 2)

        @pl.when(i >= 2)
        def _():
            pltpu.make_async_copy(st_v.at[pl.ds(slot * D, D), :],
                                  ent2.at[pl.ds(0, D), :],
                                  osems.at[slot]).wait()

        so = islot * D
        sto = slot * D
        mo = slot * (NSUB * L * L)

        @plsc.parallel_loop(0, NSUB, 1, unroll=8)
        def _trA(sub):
            dc = lax.div(sub, CH // L)
            ec = lax.rem(sub, CH // L)
            mbase = mo + sub * (L * L)
            for i in range(L):
                vals = in_v[so + dc * L + i,
                            pl.ds(pl.multiple_of(ec * L, L), L)]
                skew = i * L + jnp.bitwise_and(lanes + i, L - 1)
                plsc.store_scatter(m_v, [mbase + skew], vals)

        @plsc.parallel_loop(0, NSUB, 1, unroll=8)
        def _trB(sub):
            dc = lax.div(sub, CH // L)
            ec = lax.rem(sub, CH // L)
            mbase = mo + sub * (L * L)
            for j in range(L):
                rd = lanes * L + jnp.bitwise_and(lanes + j, L - 1)
                g = plsc.load_gather(m_v, [mbase + rd])
                st_v[sto + ec * (L // 2) + (j >> 1),
                     pl.ds(pl.multiple_of((j & 1) * D + dc * L, L), L)] = g

        is_ent = i < nent

        @pl.when(is_ent)
        def _():
            dst = pl.multiple_of((wid + i * NW) * D, D)
            pltpu.async_copy(st_v.at[pl.ds(slot * D, D), :],
                             ent2.at[pl.ds(dst, D), :], osems.at[slot])

        @pl.when(jnp.logical_not(is_ent))
        def _():
            dst = pl.multiple_of((wid - REL_W) * D, D)
            pltpu.async_copy(st_v.at[pl.ds(slot * D, D), :],
                             rel2.at[pl.ds(dst, D), :], osems.at[slot])

        return carry

    lax.fori_loop(0, nunits, body, 0)

    pltpu.make_async_copy(st_v.at[pl.ds(0, D), :], ent2.at[pl.ds(0, D), :],
                          osems.at[lax.rem(nunits - 1, 2)]).wait()

    @pl.when(nunits >= 2)
    def _():
        pltpu.make_async_copy(st_v.at[pl.ds(0, D), :],
                              ent2.at[pl.ds(0, D), :],
                              osems.at[lax.rem(nunits, 2)]).wait()


def _score_body(head_hbm, rel_hbm, tail_hbm, ent_hbm, relemb_hbm, out_hbm,
                hidx_v, ridx_v, tidx_v, hrow_v, rrow_v, trow_v,
                hbuf_v, rbuf_v, tbuf_v, q_v, out_v, sems):
    wid = lax.axis_index("s") * NC + lax.axis_index("c")
    base = wid * BPW

    pltpu.sync_copy(head_hbm.at[pl.ds(base, BPW)], hidx_v)
    pltpu.sync_copy(rel_hbm.at[pl.ds(base, BPW)], ridx_v)
    pltpu.sync_copy(tail_hbm.at[pl.ds(base, BPW)], tidx_v)

    for v in range(BPW // L):
        hrow_v[pl.ds(v * L, L)] = jnp.right_shift(hidx_v[pl.ds(v * L, L)], 1)
        rrow_v[pl.ds(v * L, L)] = jnp.right_shift(ridx_v[pl.ds(v * L, L)], 1)
        trow_v[pl.ds(v * L, L)] = jnp.right_shift(tidx_v[pl.ds(v * L, L)], 1)

    def issue(k):
        slot = lax.rem(k, 2)
        o = slot * CH
        pltpu.async_copy(ent_hbm.at[hrow_v.at[pl.ds(k * CH, CH)]],
                         hbuf_v.at[pl.ds(o, CH)], sems.at[slot])
        pltpu.async_copy(relemb_hbm.at[rrow_v.at[pl.ds(k * CH, CH)]],
                         rbuf_v.at[pl.ds(o, CH)], sems.at[slot])
        pltpu.async_copy(ent_hbm.at[trow_v.at[pl.ds(k * CH, CH)]],
                         tbuf_v.at[pl.ds(o, CH)], sems.at[slot])

    def drain(k):
        slot = lax.rem(k, 2)
        o = slot * CH
        pltpu.make_async_copy(ent_hbm.at[hrow_v.at[pl.ds(0, CH)]],
                              hbuf_v.at[pl.ds(o, CH)], sems.at[slot]).wait()
        pltpu.make_async_copy(ent_hbm.at[hrow_v.at[pl.ds(0, CH)]],
                              rbuf_v.at[pl.ds(o, CH)], sems.at[slot]).wait()
        pltpu.make_async_copy(ent_hbm.at[hrow_v.at[pl.ds(0, CH)]],
                              tbuf_v.at[pl.ds(o, CH)], sems.at[slot]).wait()

    issue(0)

    def chunk(k, carry):
        drain(k)

        @pl.when(k + 1 < NCH)
        def _():
            issue(k + 1)

        slot = lax.rem(k, 2)
        o = slot * CH

        def grp(gg, carry2):
            b0 = k * CH + gg * L
            vh = jnp.bitwise_and(hidx_v[pl.ds(b0, L)], 1) * D
            vr = jnp.bitwise_and(ridx_v[pl.ds(b0, L)], 1) * D
            vt = jnp.bitwise_and(tidx_v[pl.ds(b0, L)], 1) * D
            for j in range(L):
                i = gg * L + j
                hh, rh, th = vh[j], vr[j], vt[j]
                q = (hbuf_v[o + i, pl.ds(hh, L)] * rbuf_v[o + i, pl.ds(rh, L)]
                     * tbuf_v[o + i, pl.ds(th, L)])
                for c in range(L, D, L):
                    q += (hbuf_v[o + i, pl.ds(hh + c, L)]
                          * rbuf_v[o + i, pl.ds(rh + c, L)]
                          * tbuf_v[o + i, pl.ds(th + c, L)])
                q_v[pl.ds((b0 + j) * L, L)] = q
            return carry2

        lax.fori_loop(0, CH // L, grp, 0)
        return carry

    lax.fori_loop(0, NCH, chunk, 0)

    @plsc.parallel_loop(0, BPW // L, 1, unroll=2)
    def _rowB(g):
        rows = g * L + jnp.arange(L, dtype=jnp.int32)
        acc = plsc.load_gather(q_v, [rows * L])
        for l in range(1, L):
            acc += plsc.load_gather(q_v, [rows * L + l])
        out_v[pl.ds(g * L, L)] = acc

    pltpu.sync_copy(out_v, out_hbm.at[pl.ds(base, BPW)])


@jax.jit
def _distmult(head, relation, tail, ent_t, rel_t):
    mesh = plsc.VectorSubcoreMesh(core_axis_name="c", subcore_axis_name="s")
    ent2, rel2 = pl.kernel(
        _transpose_body,
        out_type=(jax.ShapeDtypeStruct((ENT2_ROWS, W2), jnp.float32),
                  jax.ShapeDtypeStruct((REL2_ROWS, W2), jnp.float32)),
        mesh=mesh,
        scratch_types=[
            pltpu.VMEM((4 * D, CH), jnp.float32),
            pltpu.VMEM((2 * D, W2), jnp.float32),
            pltpu.VMEM((2 * NSUB * L * L,), jnp.float32),
            pltpu.SemaphoreType.DMA((4,)),
            pltpu.SemaphoreType.DMA((2,)),
        ],
        compiler_params=pltpu.CompilerParams(
            needs_layout_passes=False, use_tc_tiling_on_sc=True),
    )(ent_t, rel_t)

    return pl.kernel(
        _score_body,
        out_type=jax.ShapeDtypeStruct((B,), jnp.float32),
        mesh=mesh,
        scratch_types=[
            pltpu.VMEM((BPW,), jnp.int32),
            pltpu.VMEM((BPW,), jnp.int32),
            pltpu.VMEM((BPW,), jnp.int32),
            pltpu.VMEM((BPW,), jnp.int32),
            pltpu.VMEM((BPW,), jnp.int32),
            pltpu.VMEM((BPW,), jnp.int32),
            pltpu.VMEM((2 * CH, W2), jnp.float32),
            pltpu.VMEM((2 * CH, W2), jnp.float32),
            pltpu.VMEM((2 * CH, W2), jnp.float32),
            pltpu.VMEM((BPW * L,), jnp.float32),
            pltpu.VMEM((BPW,), jnp.float32),
            pltpu.SemaphoreType.DMA((2,)),
        ],
        compiler_params=pltpu.CompilerParams(
            needs_layout_passes=False, use_tc_tiling_on_sc=True),
    )(head, relation, tail, ent2, rel2)


def kernel(head, relation, tail, entity_emb, relation_emb):
    rel_t = jnp.pad(relation_emb.T, ((0, 0), (0, RPAD - N_REL)))
    return _distmult(head.astype(jnp.int32), relation.astype(jnp.int32),
                     tail.astype(jnp.int32), entity_emb.T, rel_t)
